# baseline (speedup 1.0000x reference)
import functools

import jax
import jax.numpy as jnp
from jax.experimental import pallas as pl
from jax.experimental.pallas import tpu as pltpu

IN_CH = 256
HID = 256
NUM_CLASSES = 4


def _block_kernel(x_ref, w1_ref, b1_ref, part_ref, *, blk, seq_len):
    k = pl.program_id(0)

    xb = x_ref[...]
    ones = jnp.ones((8, blk), jnp.float32)

    def emit(y):
        h = jnp.maximum(
            jnp.dot(y, w1_ref[...], preferred_element_type=jnp.float32)
            + b1_ref[...], 0.0)
        part_ref[...] = jnp.dot(ones, h, preferred_element_type=jnp.float32)

    @pl.when(k == 0)
    def _chain_block():
        idx = jax.lax.broadcasted_iota(jnp.int32, xb.shape, 0)
        has_prev = idx > 0
        has_next = idx < (seq_len - 1)
        prev = pltpu.roll(xb, 1, 0)
        nxt = pltpu.roll(xb, blk - 1, 0)
        zero = jnp.zeros_like(xb)
        nbr = jnp.where(has_prev, prev, zero) + jnp.where(has_next, nxt, zero)
        inv_deg = jnp.where(jnp.logical_and(has_prev, has_next), 0.5, 1.0)
        emit(xb + nbr * inv_deg)

    @pl.when(k != 0)
    def _plain_block():
        emit(xb)


def _reduce_kernel(part_ref, w2_ref, b2_ref, out_ref, *, n_total, rep):
    parts = part_ref[...]
    ones = jnp.ones((8, parts.shape[0]), jnp.float32)
    total = jnp.dot(ones, parts, preferred_element_type=jnp.float32)
    pooled = total / jnp.float32(n_total * rep)
    out_ref[...] = (jnp.dot(pooled, w2_ref[...],
                            preferred_element_type=jnp.float32)
                    + b2_ref[...])


def kernel(x, W1, b1, W2, b2):
    batch, seq_len, d = x.shape
    xf = x.reshape(-1, d)
    n = xf.shape[0]
    blk = seq_len
    nblocks = n // blk
    rep = 8

    hid = W1.shape[1]
    ncls = W2.shape[1]
    w2p = jnp.zeros((hid, 128), W2.dtype).at[:, :ncls].set(W2)
    b2p = jnp.zeros((128,), b2.dtype).at[:ncls].set(b2)

    parts = pl.pallas_call(
        functools.partial(_block_kernel, blk=blk, seq_len=seq_len),
        grid=(nblocks,),
        in_specs=[
            pl.BlockSpec((blk, d), lambda k: (k, 0)),
            pl.BlockSpec((d, hid), lambda k: (0, 0)),
            pl.BlockSpec((hid,), lambda k: (0,)),
        ],
        out_specs=pl.BlockSpec((rep, hid), lambda k: (k, 0)),
        out_shape=jax.ShapeDtypeStruct((nblocks * rep, hid), jnp.float32),
        compiler_params=pltpu.CompilerParams(
            dimension_semantics=("parallel",)),
    )(xf, W1, b1)

    out = pl.pallas_call(
        functools.partial(_reduce_kernel, n_total=n, rep=rep),
        out_shape=jax.ShapeDtypeStruct((8, 128), jnp.float32),
    )(parts, w2p, b2p)
    return out[0:1, :ncls]


# speedup vs baseline: 1.0326x; 1.0326x over previous
import functools

import jax
import jax.numpy as jnp
from jax.experimental import pallas as pl
from jax.experimental.pallas import tpu as pltpu

IN_CH = 256
HID = 256
NUM_CLASSES = 4


def _fused_kernel(x_ref, w1_ref, b1_ref, w2_ref, b2_ref, out_ref, acc_ref,
                  *, blk, seq_len, n_total):
    k = pl.program_id(0)
    nblocks = pl.num_programs(0)

    @pl.when(k == 0)
    def _init():
        acc_ref[...] = jnp.zeros_like(acc_ref)

    xb = x_ref[...]
    ones = jnp.ones((8, blk), jnp.float32)

    def accumulate(y):
        h = jnp.maximum(
            jnp.dot(y.astype(jnp.bfloat16), w1_ref[...].astype(jnp.bfloat16),
                    preferred_element_type=jnp.float32)
            + b1_ref[...], 0.0)
        acc_ref[...] += jnp.dot(ones, h, preferred_element_type=jnp.float32)

    @pl.when(k == 0)
    def _chain_block():
        idx = jax.lax.broadcasted_iota(jnp.int32, xb.shape, 0)
        has_prev = idx > 0
        has_next = idx < (seq_len - 1)
        prev = pltpu.roll(xb, 1, 0)
        nxt = pltpu.roll(xb, blk - 1, 0)
        zero = jnp.zeros_like(xb)
        nbr = jnp.where(has_prev, prev, zero) + jnp.where(has_next, nxt, zero)
        inv_deg = jnp.where(jnp.logical_and(has_prev, has_next), 0.5, 1.0)
        accumulate(xb + nbr * inv_deg)

    @pl.when(k != 0)
    def _plain_block():
        accumulate(xb)

    @pl.when(k == nblocks - 1)
    def _final():
        pooled = acc_ref[...] / jnp.float32(n_total)
        logits = (jnp.dot(pooled, w2_ref[...],
                          preferred_element_type=jnp.float32)
                  + b2_ref[...])
        out_ref[...] = logits


def kernel(x, W1, b1, W2, b2):
    batch, seq_len, d = x.shape
    xf = x.reshape(-1, d)
    n = xf.shape[0]
    blk = seq_len
    nblocks = n // blk

    hid = W1.shape[1]
    ncls = W2.shape[1]
    w2p = jnp.zeros((hid, 128), W2.dtype).at[:, :ncls].set(W2)
    b2p = jnp.zeros((128,), b2.dtype).at[:ncls].set(b2)

    out = pl.pallas_call(
        functools.partial(_fused_kernel, blk=blk, seq_len=seq_len, n_total=n),
        grid=(nblocks,),
        in_specs=[
            pl.BlockSpec((blk, d), lambda k: (k, 0)),
            pl.BlockSpec((d, hid), lambda k: (0, 0)),
            pl.BlockSpec((hid,), lambda k: (0,)),
            pl.BlockSpec((hid, 128), lambda k: (0, 0)),
            pl.BlockSpec((128,), lambda k: (0,)),
        ],
        out_specs=pl.BlockSpec((8, 128), lambda k: (0, 0)),
        out_shape=jax.ShapeDtypeStruct((8, 128), jnp.float32),
        scratch_shapes=[pltpu.VMEM((8, hid), jnp.float32)],
    )(xf, W1, b1, w2p, b2p)
    return out[0:1, :ncls]


# speedup vs baseline: 1.1029x; 1.0681x over previous
import functools

import jax
import jax.numpy as jnp
from jax.experimental import pallas as pl
from jax.experimental.pallas import tpu as pltpu

IN_CH = 256
HID = 256
NUM_CLASSES = 4


def _fused_kernel(x_ref, w1_ref, b1_ref, w2_ref, b2_ref, out_ref, acc_ref,
                  *, blk, seq_len, n_total):
    k = pl.program_id(0)
    nblocks = pl.num_programs(0)

    @pl.when(k == 0)
    def _init():
        acc_ref[...] = jnp.zeros_like(acc_ref)

    xb = x_ref[...]
    ones = jnp.ones((8, blk), jnp.float32)

    def accumulate(y):
        h = jnp.maximum(
            jnp.dot(y, w1_ref[...], preferred_element_type=jnp.float32)
            + b1_ref[...], 0.0)
        acc_ref[...] += jnp.dot(ones, h, preferred_element_type=jnp.float32)

    @pl.when(k == 0)
    def _chain_block():
        idx = jax.lax.broadcasted_iota(jnp.int32, xb.shape, 0)
        has_prev = jnp.logical_and(idx > 0, idx < seq_len)
        has_next = idx < (seq_len - 1)
        prev = pltpu.roll(xb, 1, 0)
        nxt = pltpu.roll(xb, blk - 1, 0)
        zero = jnp.zeros_like(xb)
        nbr = jnp.where(has_prev, prev, zero) + jnp.where(has_next, nxt, zero)
        inv_deg = jnp.where(jnp.logical_and(has_prev, has_next), 0.5, 1.0)
        accumulate(xb + nbr * inv_deg)

    @pl.when(k != 0)
    def _plain_block():
        accumulate(xb)

    @pl.when(k == nblocks - 1)
    def _final():
        pooled = acc_ref[...] / jnp.float32(n_total)
        logits = (jnp.dot(pooled, w2_ref[...],
                          preferred_element_type=jnp.float32)
                  + b2_ref[...])
        out_ref[...] = logits


def kernel(x, W1, b1, W2, b2):
    batch, seq_len, d = x.shape
    xf = x.reshape(-1, d)
    n = xf.shape[0]
    blk = 2 * seq_len
    nblocks = n // blk

    hid = W1.shape[1]
    ncls = W2.shape[1]
    w2p = jnp.zeros((hid, 128), W2.dtype).at[:, :ncls].set(W2)
    b2p = jnp.zeros((128,), b2.dtype).at[:ncls].set(b2)

    out = pl.pallas_call(
        functools.partial(_fused_kernel, blk=blk, seq_len=seq_len, n_total=n),
        grid=(nblocks,),
        in_specs=[
            pl.BlockSpec((blk, d), lambda k: (k, 0)),
            pl.BlockSpec((d, hid), lambda k: (0, 0)),
            pl.BlockSpec((hid,), lambda k: (0,)),
            pl.BlockSpec((hid, 128), lambda k: (0, 0)),
            pl.BlockSpec((128,), lambda k: (0,)),
        ],
        out_specs=pl.BlockSpec((8, 128), lambda k: (0, 0)),
        out_shape=jax.ShapeDtypeStruct((8, 128), jnp.float32),
        scratch_shapes=[pltpu.VMEM((8, hid), jnp.float32)],
    )(xf, W1, b1, w2p, b2p)
    return out[0:1, :ncls]
